# baseline (speedup 1.0000x reference)
import functools

import jax
import jax.numpy as jnp
from jax import lax
from jax.experimental import pallas as pl
from jax.experimental.pallas import tpu as pltpu
from jax.experimental.pallas import tpu_sc as plsc

N_PATCHES = 200
AH = AW = 128
H = W = 2048
C = 3
PATCH = 32
SY = H // AH
NC, NS = 2, 16
NW = NC * NS
UNITS = 2 * N_PATCHES * C
UPW = 38
UPAD = NW * UPW
BLKW = 256
KPAD = 256


NCH = 2
CHR = AH // NCH
SORTN = 512


def _topk_body(score_ref, p_ref, sa_ref, idx_ref):
    pos = (lax.broadcasted_iota(jnp.int32, (CHR, AW), 0) * AW
           + lax.broadcasted_iota(jnp.int32, (CHR, AW), 1))
    lane = lax.broadcasted_iota(jnp.int32, (KPAD,), 0)
    big = jnp.int32(1 << 30)
    neg = jnp.float32(-1e30)
    ninf = jnp.float32(-3e38)

    def step(j, st):
        s, kv, iv, av = st
        m = jnp.max(s)
        local = jnp.min(jnp.where(s == m, pos, big))
        hit = pos == local
        sel = lane == j
        kv = jnp.where(sel, m, kv)
        iv = jnp.where(sel, local, iv)
        return jnp.where(hit, neg, s), kv, iv, av, hit, sel

    def body(j, st):
        out = []
        for chain, pchunk in zip(st, pchunks):
            s, kv, iv, av, hit, sel = step(j, chain)
            pv = jnp.sum(jnp.where(hit, pchunk, jnp.float32(0.0)))
            av = jnp.where(sel, pv, av)
            out.append((s, kv, iv, av))
        return tuple(out)

    z_i = jnp.zeros((KPAD,), jnp.int32)
    z_f = jnp.zeros((KPAD,), jnp.float32)
    nf = jnp.full((KPAD,), ninf, jnp.float32)

    chains0 = tuple(
        (score_ref[b, pl.ds(k * CHR, CHR), :], nf, z_i, z_f)
        for b in range(2) for k in range(NCH))
    pchunks = tuple(
        p_ref[b, pl.ds(k * CHR, CHR), :]
        for b in range(2) for k in range(NCH))

    chains = lax.fori_loop(0, N_PATCHES, body, chains0)

    io_s = lax.broadcasted_iota(jnp.int32, (SORTN,), 0)
    for b in range(2):
        (_, k0, i0, a0), (_, k1, i1, a1) = chains[2 * b], chains[2 * b + 1]
        keys = jnp.concatenate([k0, k1])
        idxs = jnp.concatenate([i0, i1 + CHR * AW])
        avs = jnp.concatenate([a0, a1])
        idxs = jnp.where(keys == ninf, big, idxs)
        for stk in range(1, 10):
            for stj in range(stk - 1, -1, -1):
                jj = 1 << stj
                kk = 1 << stk
                up = (io_s & kk) == 0
                low = (io_s & jj) == 0
                okey = jnp.where(low, jnp.roll(keys, -jj),
                                 jnp.roll(keys, jj))
                oidx = jnp.where(low, jnp.roll(idxs, -jj),
                                 jnp.roll(idxs, jj))
                oav = jnp.where(low, jnp.roll(avs, -jj),
                                jnp.roll(avs, jj))
                cmp_so = jnp.logical_or(
                    keys > okey,
                    jnp.logical_and(keys == okey, idxs < oidx))
                want = cmp_so == (low == up)
                keys = jnp.where(want, keys, okey)
                idxs = jnp.where(want, idxs, oidx)
                avs = jnp.where(want, avs, oav)
        idx_ref[b, 0] = idxs[:KPAD]
        sa_ref[b, 0] = avs[:KPAD]


def _topk_call(score, p):
    return pl.pallas_call(
        _topk_body,
        out_shape=[jax.ShapeDtypeStruct((2, 1, KPAD), jnp.float32),
                   jax.ShapeDtypeStruct((2, 1, KPAD), jnp.int32)],
    )(score, p)


@functools.cache
def _make_gather():
    mesh = plsc.VectorSubcoreMesh(core_axis_name="c", subcore_axis_name="s")

    @functools.partial(
        pl.kernel,
        mesh=mesh,
        out_type=jax.ShapeDtypeStruct((2, N_PATCHES, C, PATCH, PATCH),
                                      jnp.float32),
        compiler_params=pltpu.CompilerParams(use_tc_tiling_on_sc=True),
        scratch_types=[
            pltpu.VMEM((UPAD // 8, 128), jnp.int32),
            pltpu.VMEM((PATCH, BLKW), jnp.float32),
            pltpu.VMEM((PATCH, BLKW), jnp.float32),
            pltpu.VMEM((PATCH, PATCH), jnp.float32),
            pltpu.VMEM((PATCH, PATCH), jnp.float32),
            pltpu.SemaphoreType.DMA,
            pltpu.SemaphoreType.DMA,
            pltpu.SemaphoreType.DMA,
            pltpu.SemaphoreType.DMA,
        ],
    )
    def gather_k(wsi_hbm, desc_hbm, out_hbm, desc_v, buf0, buf1,
                 pbuf0, pbuf1, sr0, sr1, sw0, sw1):
        wid = lax.axis_index("s") * NC + lax.axis_index("c")
        pltpu.sync_copy(desc_hbm, desc_v)
        bufs = (buf0, buf1)
        pbufs = (pbuf0, pbuf1)
        srs = (sr0, sr1)
        sws = (sw0, sw1)

        def fields(t):
            u = t * NW + wid
            r = u // 8
            c0 = pl.multiple_of((u - r * 8) * 16, 16)
            v = desc_v[r, pl.ds(c0, 16)]
            return v[0], v[1], v[2], v[3], v[4], v[5]

        def start_read(t, buf, sem):
            row0, xa, _, _, _, _ = fields(t)
            row0 = pl.multiple_of(row0, 16)
            xa = pl.multiple_of(xa, 128)
            return pltpu.async_copy(
                wsi_hbm.at[pl.ds(row0, PATCH), pl.ds(xa, BLKW)], buf, sem)

        reads = [start_read(0, buf0, sr0), start_read(1, buf1, sr1)]
        writes = [None, None]
        for t in range(UPW):
            pipe = t % 2
            buf = bufs[pipe]
            pbuf = pbufs[pipe]
            reads[pipe].wait()
            if writes[pipe] is not None:
                writes[pipe].wait()
            _, _, xoff, ob, on, oc = fields(t)
            xoff = pl.multiple_of(xoff, 16)
            for r in range(PATCH):
                for h in range(2):
                    pbuf[r, pl.ds(h * 16, 16)] = (
                        buf[r, pl.ds(xoff + h * 16, 16)])
            writes[pipe] = pltpu.async_copy(
                pbuf, out_hbm.at[ob, on, oc], sws[pipe])
            if t + 2 < UPW:
                reads[pipe] = start_read(t + 2, buf, srs[pipe])
        writes[0].wait()
        writes[1].wait()

    return gather_k


def kernel(x_low, x_high, attention, WSI):
    B = attention.shape[0]
    flat = attention.reshape(B, -1)
    p = flat / jnp.sum(flat, axis=-1, keepdims=True)
    logp = jnp.log(p + 1e-12)
    u = jax.random.uniform(jax.random.key(42), flat.shape,
                           minval=1e-9, maxval=1.0)
    gumbel = -jnp.log(-jnp.log(u))
    score = logp + gumbel
    sa_pad, idx_pad = _topk_call(score.reshape(B, AH, AW),
                                 p.reshape(B, AH, AW))

    cell = idx_pad[:, 0, :N_PATCHES]
    ys = cell // AW
    xs = cell % AW
    y0 = jnp.minimum(ys * SY, H - PATCH)
    x0 = jnp.minimum(xs * SY, W - PATCH)
    xa = jnp.minimum((x0 // 128) * 128, W - BLKW)
    xoff = (x0 - xa)[:, :, None]
    xa = xa[:, :, None]
    cc = jnp.arange(C, dtype=jnp.int32)[None, None, :]
    bb = jnp.arange(B, dtype=jnp.int32)[:, None, None]
    nn = jnp.arange(N_PATCHES, dtype=jnp.int32)[None, :, None]
    row0 = (bb * C + cc) * H + y0[:, :, None]
    zz = jnp.zeros((B, N_PATCHES, C), jnp.int32)
    fields = jnp.stack(
        [row0, xa + zz, xoff + zz, bb + zz, nn + zz, cc + zz],
        axis=-1).reshape(UNITS, 6).astype(jnp.int32)
    fields = jnp.concatenate(
        [fields, jnp.broadcast_to(fields[:1], (UPAD - UNITS, 6))], axis=0)
    desc = jnp.pad(fields, ((0, 0), (0, 10))).reshape(UPAD // 8, 128)

    patches = _make_gather()(WSI.reshape(B * C * H, W), desc)
    return patches, sa_pad[:, 0, :N_PATCHES]


# speedup vs baseline: 2.1202x; 2.1202x over previous
import functools

import jax
import jax.numpy as jnp
from jax import lax
from jax.experimental import pallas as pl
from jax.experimental.pallas import tpu as pltpu
from jax.experimental.pallas import tpu_sc as plsc

N_PATCHES = 200
AH = AW = 128
H = W = 2048
C = 3
PATCH = 32
SY = H // AH
NC, NS = 2, 16
NW = NC * NS
UNITS = 2 * N_PATCHES * C
UPW = 38
UPAD = NW * UPW
BLKW = 256
KPAD = 256


NCH = 4
CHR = AH // NCH


def _topk_body(score_ref, p_ref, sa_ref, idx_ref):
    pos = (lax.broadcasted_iota(jnp.int32, (CHR, AW), 0) * AW
           + lax.broadcasted_iota(jnp.int32, (CHR, AW), 1))
    lane = lax.broadcasted_iota(jnp.int32, (KPAD,), 0)
    big = jnp.int32(1 << 30)
    neg = jnp.float32(-1e30)

    def step(j, chunks, pcs, cms, idxv, sav):
        m = jnp.maximum(jnp.maximum(cms[0], cms[1]),
                        jnp.maximum(cms[2], cms[3]))
        p0 = cms[0] == m
        p1 = jnp.logical_and(jnp.logical_not(p0), cms[1] == m)
        p01 = jnp.logical_or(p0, p1)
        p2 = jnp.logical_and(jnp.logical_not(p01), cms[2] == m)
        preds = (p0, p1, p2)
        sel = jnp.where(p0, chunks[0],
                        jnp.where(p1, chunks[1],
                                  jnp.where(p2, chunks[2], chunks[3])))
        psel = jnp.where(p0, pcs[0],
                         jnp.where(p1, pcs[1],
                                   jnp.where(p2, pcs[2], pcs[3])))
        base = jnp.where(p0, jnp.int32(0),
                         jnp.where(p1, jnp.int32(CHR * AW),
                                   jnp.where(p2, jnp.int32(2 * CHR * AW),
                                             jnp.int32(3 * CHR * AW))))
        local = jnp.min(jnp.where(sel == m, pos, big))
        hit = pos == local
        pv = jnp.sum(jnp.where(hit, psel, jnp.float32(0.0)))
        upd = jnp.where(hit, neg, sel)
        mx = jnp.max(upd)
        new_chunks = []
        new_cms = []
        for k in range(NCH):
            pk = preds[k] if k < 3 else jnp.logical_not(
                jnp.logical_or(p01, p2))
            new_chunks.append(jnp.where(pk, upd, chunks[k]))
            new_cms.append(jnp.where(pk, mx, cms[k]))
        idxv = jnp.where(lane == j, base + local, idxv)
        sav = jnp.where(lane == j, pv, sav)
        return tuple(new_chunks), tuple(new_cms), idxv, sav

    def body(j, st):
        c0, m0, i0, a0, c1, m1, i1, a1 = st
        c0, m0, i0, a0 = step(j, c0, _pc(0), m0, i0, a0)
        c1, m1, i1, a1 = step(j, c1, _pc(1), m1, i1, a1)
        return c0, m0, i0, a0, c1, m1, i1, a1

    def _pc(b):
        return tuple(p_ref[b, pl.ds(k * CHR, CHR), :] for k in range(NCH))

    z_i = jnp.zeros((KPAD,), jnp.int32)
    z_f = jnp.zeros((KPAD,), jnp.float32)

    def init(b):
        chunks = tuple(score_ref[b, pl.ds(k * CHR, CHR), :]
                       for k in range(NCH))
        cms = tuple(jnp.max(c) for c in chunks)
        return chunks, cms

    c0, m0 = init(0)
    c1, m1 = init(1)
    _, _, i0, a0, _, _, i1, a1 = lax.fori_loop(
        0, N_PATCHES, body, (c0, m0, z_i, z_f, c1, m1, z_i, z_f))

    idx_ref[0, 0] = i0
    idx_ref[1, 0] = i1
    sa_ref[0, 0] = a0
    sa_ref[1, 0] = a1


def _topk_call(score, p):
    return pl.pallas_call(
        _topk_body,
        out_shape=[jax.ShapeDtypeStruct((2, 1, KPAD), jnp.float32),
                   jax.ShapeDtypeStruct((2, 1, KPAD), jnp.int32)],
    )(score, p)


@functools.cache
def _make_gather():
    mesh = plsc.VectorSubcoreMesh(core_axis_name="c", subcore_axis_name="s")

    @functools.partial(
        pl.kernel,
        mesh=mesh,
        out_type=jax.ShapeDtypeStruct((2, N_PATCHES, C, PATCH, PATCH),
                                      jnp.float32),
        compiler_params=pltpu.CompilerParams(use_tc_tiling_on_sc=True),
        scratch_types=[
            pltpu.VMEM((UPAD // 8, 128), jnp.int32),
            pltpu.VMEM((PATCH, BLKW), jnp.float32),
            pltpu.VMEM((PATCH, BLKW), jnp.float32),
            pltpu.VMEM((PATCH, PATCH), jnp.float32),
            pltpu.VMEM((PATCH, PATCH), jnp.float32),
            pltpu.SemaphoreType.DMA,
            pltpu.SemaphoreType.DMA,
            pltpu.SemaphoreType.DMA,
            pltpu.SemaphoreType.DMA,
        ],
    )
    def gather_k(wsi_hbm, desc_hbm, out_hbm, desc_v, buf0, buf1,
                 pbuf0, pbuf1, sr0, sr1, sw0, sw1):
        wid = lax.axis_index("s") * NC + lax.axis_index("c")
        pltpu.sync_copy(desc_hbm, desc_v)
        bufs = (buf0, buf1)
        pbufs = (pbuf0, pbuf1)
        srs = (sr0, sr1)
        sws = (sw0, sw1)

        def fields(t):
            u = t * NW + wid
            r = u // 8
            c0 = pl.multiple_of((u - r * 8) * 16, 16)
            v = desc_v[r, pl.ds(c0, 16)]
            return v[0], v[1], v[2], v[3], v[4], v[5]

        def start_read(t, buf, sem):
            row0, xa, _, _, _, _ = fields(t)
            row0 = pl.multiple_of(row0, 16)
            xa = pl.multiple_of(xa, 128)
            return pltpu.async_copy(
                wsi_hbm.at[pl.ds(row0, PATCH), pl.ds(xa, BLKW)], buf, sem)

        reads = [start_read(0, buf0, sr0), start_read(1, buf1, sr1)]
        writes = [None, None]
        for t in range(UPW):
            pipe = t % 2
            buf = bufs[pipe]
            pbuf = pbufs[pipe]
            reads[pipe].wait()
            if writes[pipe] is not None:
                writes[pipe].wait()
            _, _, xoff, ob, on, oc = fields(t)
            xoff = pl.multiple_of(xoff, 16)
            for r in range(PATCH):
                for h in range(2):
                    pbuf[r, pl.ds(h * 16, 16)] = (
                        buf[r, pl.ds(xoff + h * 16, 16)])
            writes[pipe] = pltpu.async_copy(
                pbuf, out_hbm.at[ob, on, oc], sws[pipe])
            if t + 2 < UPW:
                reads[pipe] = start_read(t + 2, buf, srs[pipe])
        writes[0].wait()
        writes[1].wait()

    return gather_k


def kernel(x_low, x_high, attention, WSI):
    B = attention.shape[0]
    flat = attention.reshape(B, -1)
    p = flat / jnp.sum(flat, axis=-1, keepdims=True)
    logp = jnp.log(p + 1e-12)
    u = jax.random.uniform(jax.random.key(42), flat.shape,
                           minval=1e-9, maxval=1.0)
    gumbel = -jnp.log(-jnp.log(u))
    score = logp + gumbel
    sa_pad, idx_pad = _topk_call(score.reshape(B, AH, AW),
                                 p.reshape(B, AH, AW))

    cell = idx_pad[:, 0, :N_PATCHES]
    ys = cell // AW
    xs = cell % AW
    y0 = jnp.minimum(ys * SY, H - PATCH)
    x0 = jnp.minimum(xs * SY, W - PATCH)
    xa = jnp.minimum((x0 // 128) * 128, W - BLKW)
    xoff = (x0 - xa)[:, :, None]
    xa = xa[:, :, None]
    cc = jnp.arange(C, dtype=jnp.int32)[None, None, :]
    bb = jnp.arange(B, dtype=jnp.int32)[:, None, None]
    nn = jnp.arange(N_PATCHES, dtype=jnp.int32)[None, :, None]
    row0 = (bb * C + cc) * H + y0[:, :, None]
    zz = jnp.zeros((B, N_PATCHES, C), jnp.int32)
    fields = jnp.stack(
        [row0, xa + zz, xoff + zz, bb + zz, nn + zz, cc + zz],
        axis=-1).reshape(UNITS, 6).astype(jnp.int32)
    fields = jnp.concatenate(
        [fields, jnp.broadcast_to(fields[:1], (UPAD - UNITS, 6))], axis=0)
    desc = jnp.pad(fields, ((0, 0), (0, 10))).reshape(UPAD // 8, 128)

    patches = _make_gather()(WSI.reshape(B * C * H, W), desc)
    return patches, sa_pad[:, 0, :N_PATCHES]
